# Initial kernel scaffold; baseline (speedup 1.0000x reference)
import jax
import jax.numpy as jnp
from jax.experimental import pallas as pl


def kernel(x, edge_index, W_self0, W_neigh0, b0, W_self1, W_neigh1, b1, W_fc, b_fc):
    raise NotImplementedError("write your pallas kernel here")


# speedup vs baseline: 5.2717x; 5.2717x over previous
import functools

import jax
import jax.numpy as jnp
from jax import lax
from jax.experimental import pallas as pl
from jax.experimental.pallas import tpu as pltpu
from jax.experimental.pallas import tpu_sc as plsc

N = 10000
E = 320000
D = 128
H = 128
C = 64

NC = 2
NS = 16
NPAD = 10240
EPC = E // NC
EPT = EPC // NS
CHUNK = 80
NCHUNK = EPT // CHUNK
RPT = NPAD // NS
DEGW = 16

f32 = jnp.float32


def _make_agg(with_deg: bool, stage: int = 3):
  mesh = plsc.VectorSubcoreMesh(
      core_axis_name="c", subcore_axis_name="s",
      num_cores=NC, num_subcores=NS)

  out_type = [jax.ShapeDtypeStruct((NC * NPAD, D), f32)]
  if with_deg:
    out_type.append(jax.ShapeDtypeStruct((NC * NPAD, DEGW), f32))

  scratch = [
      pltpu.VMEM((CHUNK,), jnp.int32),
      pltpu.VMEM((CHUNK,), jnp.int32),
      pltpu.VMEM((CHUNK, D), f32),
      pltpu.VMEM_SHARED((NPAD, D), f32),
      pltpu.SemaphoreType.DMA,
      pltpu.SemaphoreType.DMA,
  ]
  if with_deg:
    scratch += [
        pltpu.VMEM((CHUNK, DEGW), f32),
        pltpu.VMEM_SHARED((NPAD, DEGW), f32),
        pltpu.SemaphoreType.DMA,
    ]

  def body(*refs):
    if with_deg:
      (feat, srcs, dsts, zrows, zdeg, out_acc, out_deg,
       sidx, didx, rows, acc, sem_g, sem_s,
       ones, dacc, sem_d) = refs
    else:
      (feat, srcs, dsts, zrows, out_acc,
       sidx, didx, rows, acc, sem_g, sem_s) = refs

    c = lax.axis_index("c")
    s = lax.axis_index("s")
    rbase = s * RPT

    pltpu.sync_copy(zrows, rows)
    if stage >= -1:
      for t in range(RPT // CHUNK):
        pltpu.sync_copy(rows, acc.at[pl.ds(rbase + t * CHUNK, CHUNK)])
    if with_deg:
      pltpu.sync_copy(zdeg, ones)
      if stage >= -1:
        for t in range(RPT // CHUNK):
          pltpu.sync_copy(ones, dacc.at[pl.ds(rbase + t * CHUNK, CHUNK)])
      def fill(r, carry):
        ones[r, :] = jnp.ones((DEGW,), f32)
        return carry
      lax.fori_loop(0, CHUNK, fill, 0)
    if stage >= 0:
      plsc.subcore_barrier()

    ebase = c * EPC + s * EPT

    def chunk(j, carry):
      off = ebase + j * CHUNK
      pltpu.sync_copy(srcs.at[pl.ds(off, CHUNK)], sidx)
      pltpu.sync_copy(dsts.at[pl.ds(off, CHUNK)], didx)
      if stage >= 2:
        pltpu.async_copy(feat.at[sidx], rows, sem_g).wait()
      if stage >= 3:
        cp = pltpu.async_copy(rows, acc.at[didx], sem_s, add=True)
        if with_deg:
          pltpu.async_copy(ones, dacc.at[didx], sem_d, add=True).wait()
        cp.wait()
      return carry

    if stage >= 1:
      lax.fori_loop(0, NCHUNK, chunk, 0)
    if stage >= 0:
      plsc.subcore_barrier()

    for t in range(RPT // CHUNK):
      if stage >= -1:
        pltpu.sync_copy(acc.at[pl.ds(rbase + t * CHUNK, CHUNK)], rows)
      pltpu.sync_copy(rows,
                      out_acc.at[pl.ds(c * NPAD + rbase + t * CHUNK, CHUNK)])
    if with_deg:
      for t in range(RPT // CHUNK):
        if stage >= -1:
          pltpu.sync_copy(dacc.at[pl.ds(rbase + t * CHUNK, CHUNK)], ones)
        pltpu.sync_copy(ones,
                        out_deg.at[pl.ds(c * NPAD + rbase + t * CHUNK, CHUNK)])

  return pl.kernel(
      body, out_type=out_type, mesh=mesh, scratch_types=scratch,
      compiler_params=pltpu.CompilerParams(use_tc_tiling_on_sc=False))


STAGE = 3
_agg_deg = _make_agg(with_deg=True, stage=STAGE)
_agg = _make_agg(with_deg=False, stage=STAGE)


ROWS = 1000


def _l1_body(x, a0, a1, d0, d1, ws, wn, b, o):
  deg = d0[...][:, 0:1] + d1[...][:, 0:1]
  inv = 1.0 / jnp.maximum(deg, 1.0)
  hn = (a0[...] + a1[...]) * inv
  s0 = jnp.dot(x[...], ws[...], preferred_element_type=f32)
  o[...] = jnp.maximum(
      s0 + jnp.dot(hn, wn[...], preferred_element_type=f32) + b[...], 0.0)


def _l2_body(h1, a0, a1, d0, d1, ws, wn, b, wfc, bfc, o):
  deg = d0[...][:, 0:1] + d1[...][:, 0:1]
  inv = 1.0 / jnp.maximum(deg, 1.0)
  hn = (a0[...] + a1[...]) * inv
  h2 = jnp.maximum(
      jnp.dot(h1[...], ws[...], preferred_element_type=f32)
      + jnp.dot(hn, wn[...], preferred_element_type=f32) + b[...], 0.0)
  o[...] = jnp.dot(h2, wfc[...], preferred_element_type=f32) + bfc[...]


def _row_spec(w):
  return pl.BlockSpec((ROWS, w), lambda i: (i, 0))


def _full_spec(shape):
  return pl.BlockSpec(shape, lambda i: tuple(0 for _ in shape))


def _tc_layer1(x, a0, a1, d0, d1, ws, wn, b):
  return pl.pallas_call(
      _l1_body,
      grid=(N // ROWS,),
      in_specs=[
          _row_spec(D), _row_spec(D), _row_spec(D),
          _row_spec(DEGW), _row_spec(DEGW),
          _full_spec((D, H)), _full_spec((D, H)), _full_spec((1, H)),
      ],
      out_specs=_row_spec(H),
      out_shape=jax.ShapeDtypeStruct((N, H), f32),
  )(x, a0, a1, d0, d1, ws, wn, b)


def _tc_layer2(h1, a0, a1, d0, d1, ws, wn, b, wfc, bfc):
  return pl.pallas_call(
      _l2_body,
      grid=(N // ROWS,),
      in_specs=[
          _row_spec(H), _row_spec(H), _row_spec(H),
          _row_spec(DEGW), _row_spec(DEGW),
          _full_spec((H, H)), _full_spec((H, H)), _full_spec((1, H)),
          _full_spec((H, C)), _full_spec((1, C)),
      ],
      out_specs=_row_spec(C),
      out_shape=jax.ShapeDtypeStruct((N, C), f32),
  )(h1, a0, a1, d0, d1, ws, wn, b, wfc, bfc)


def kernel(x, edge_index, W_self0, W_neigh0, b0, W_self1, W_neigh1, b1,
           W_fc, b_fc):
  src = edge_index[0]
  dst = edge_index[1]
  zrows = jnp.zeros((CHUNK, D), f32)
  zdeg = jnp.zeros((CHUNK, DEGW), f32)

  aggx, deg = _agg_deg(x, src, dst, zrows, zdeg)
  h1 = _tc_layer1(x, aggx[:N], aggx[NPAD:NPAD + N], deg[:N],
                  deg[NPAD:NPAD + N], W_self0, W_neigh0, b0.reshape(1, H))
  aggh, = _agg(h1, src, dst, zrows)
  out = _tc_layer2(h1, aggh[:N], aggh[NPAD:NPAD + N], deg[:N],
                   deg[NPAD:NPAD + N], W_self1, W_neigh1, b1.reshape(1, H),
                   W_fc, b_fc.reshape(1, C))
  return out
